# baseline (speedup 1.0000x reference)
import functools

import jax
import jax.numpy as jnp
from jax import lax
from jax.experimental import pallas as pl
from jax.experimental.pallas import tpu as pltpu
from jax.experimental.pallas import tpu_sc as plsc

D_MODEL = 64
SCALE = 8.0
LANES = 16
NUM_CORES = 2
NUM_SUBCORES = 16
NUM_WORKERS = NUM_CORES * NUM_SUBCORES
BBLK = 128


@functools.lru_cache(maxsize=None)
def _build_sc_gather(n_batch: int, seq_len: int):
    assert n_batch == NUM_WORKERS * BBLK
    mesh = plsc.VectorSubcoreMesh(core_axis_name="c", subcore_axis_name="s")
    pair_t = pltpu.VMEM((BBLK, 2 * D_MODEL), jnp.float32)
    outb_t = pltpu.VMEM((D_MODEL, BBLK), jnp.float32)

    @functools.partial(
        pl.kernel,
        mesh=mesh,
        out_type=jax.ShapeDtypeStruct((seq_len, D_MODEL, n_batch),
                                      jnp.float32),
        compiler_params=pltpu.CompilerParams(needs_layout_passes=False),
        scratch_types=[
            pltpu.VMEM((seq_len, BBLK), jnp.int32),
            pltpu.VMEM((seq_len, BBLK), jnp.int32),
            pair_t, pair_t,
            outb_t, outb_t,
            pltpu.SemaphoreType.DMA, pltpu.SemaphoreType.DMA,
            pltpu.SemaphoreType.DMA, pltpu.SemaphoreType.DMA,
        ],
    )
    def gather_kernel(idx2_hbm, half_hbm, table2_hbm, out_hbm,
                      idx_v, half_v, g0, g1, o0, o1,
                      gs0, gs1, os0, os1):
        gbuf = (g0, g1)
        obuf = (o0, o1)
        gsem = (gs0, gs1)
        osem = (os0, os1)
        wid = lax.axis_index("s") * NUM_CORES + lax.axis_index("c")
        b0 = wid * BBLK
        pltpu.sync_copy(idx2_hbm.at[:, pl.ds(b0, BBLK)], idx_v)
        pltpu.sync_copy(half_hbm.at[:, pl.ds(b0, BBLK)], half_v)

        rows_j = [jnp.arange(LANES, dtype=jnp.int32) + j * LANES
                  for j in range(BBLK // LANES)]

        def start_gather(t, b):
            pltpu.async_copy(table2_hbm.at[idx_v.at[t]], gbuf[b], gsem[b])

        def wait_gather(b):
            pltpu.make_async_copy(table2_hbm.at[idx_v.at[0]], gbuf[b],
                                  gsem[b]).wait()

        def start_out(t, b):
            pltpu.async_copy(obuf[b], out_hbm.at[t, :, pl.ds(b0, BBLK)],
                             osem[b])

        def wait_out(b):
            pltpu.make_async_copy(obuf[b],
                                  out_hbm.at[0, :, pl.ds(b0, BBLK)],
                                  osem[b]).wait()

        def transpose_scale(t, b):
            src, dst = gbuf[b], obuf[b]
            cols_j = [half_v[t, pl.ds(j * LANES, LANES)]
                      for j in range(BBLK // LANES)]

            for d in range(D_MODEL):
                for j in range(BBLK // LANES):
                    vals = plsc.load_gather(src, [rows_j[j], cols_j[j] + d])
                    dst[d, pl.ds(j * LANES, LANES)] = vals * SCALE

        start_gather(0, 0)
        start_gather(1, 1)

        def body(h, carry):
            t = 2 * h
            for b in range(2):
                wait_gather(b)

                @pl.when(h > 0)
                def _():
                    wait_out(b)

                transpose_scale(t + b, b)
                start_out(t + b, b)
                start_gather((t + b + 2) % seq_len, b)
            return carry

        lax.fori_loop(0, seq_len // 2, body, 0)
        wait_gather(0)
        wait_gather(1)
        wait_out(0)
        wait_out(1)

    return gather_kernel


def kernel(x, table):
    n_batch, seq_len = x.shape
    n_vocab = table.shape[0]
    xt = x.T
    idx2 = xt >> 1
    half = (xt & 1) * D_MODEL
    table2 = table.reshape(n_vocab // 2, 2 * D_MODEL)
    out = _build_sc_gather(n_batch, seq_len)(idx2, half, table2)
    return out.transpose(2, 0, 1)


# speedup vs baseline: 1.6476x; 1.6476x over previous
import functools

import jax
import jax.numpy as jnp
from jax import lax
from jax.experimental import pallas as pl
from jax.experimental.pallas import tpu as pltpu
from jax.experimental.pallas import tpu_sc as plsc

D_MODEL = 64
SCALE = 8.0
LANES = 16
NUM_CORES = 2
NUM_SUBCORES = 16
NUM_WORKERS = NUM_CORES * NUM_SUBCORES
CHUNK = 128
ROW_UNROLL = 4


@functools.lru_cache(maxsize=None)
def _build_sc_gather(n_rows: int):
    per_worker = n_rows // NUM_WORKERS
    n_chunks = per_worker // CHUNK
    assert n_chunks % 2 == 0
    mesh = plsc.VectorSubcoreMesh(core_axis_name="c", subcore_axis_name="s")
    rows_t = pltpu.VMEM((CHUNK, D_MODEL), jnp.float32)

    @functools.partial(
        pl.kernel,
        mesh=mesh,
        out_type=jax.ShapeDtypeStruct((n_rows, D_MODEL), jnp.float32),
        compiler_params=pltpu.CompilerParams(use_tc_tiling_on_sc=False),
        scratch_types=[
            pltpu.VMEM((n_chunks, CHUNK), jnp.int32),
            rows_t, rows_t,
            rows_t, rows_t,
            pltpu.SemaphoreType.DMA, pltpu.SemaphoreType.DMA,
            pltpu.SemaphoreType.DMA, pltpu.SemaphoreType.DMA,
        ],
    )
    def gather_kernel(x_hbm, table_hbm, out_hbm, idx_v,
                      g0, g1, o0, o1, gs0, gs1, os0, os1):
        gbuf = (g0, g1)
        obuf = (o0, o1)
        gsem = (gs0, gs1)
        osem = (os0, os1)
        wid = lax.axis_index("s") * NUM_CORES + lax.axis_index("c")
        chunk0 = wid * n_chunks
        pltpu.sync_copy(x_hbm.at[pl.ds(chunk0, n_chunks)], idx_v)

        def start_gather(g, b):
            pltpu.async_copy(table_hbm.at[idx_v.at[g]], gbuf[b], gsem[b])

        def wait_gather(b):
            pltpu.make_async_copy(table_hbm.at[idx_v.at[0]], gbuf[b],
                                  gsem[b]).wait()

        def start_out(g, b):
            pltpu.async_copy(obuf[b],
                             out_hbm.at[pl.ds((chunk0 + g) * CHUNK, CHUNK)],
                             osem[b])

        def wait_out(b):
            pltpu.make_async_copy(
                obuf[b], out_hbm.at[pl.ds(chunk0 * CHUNK, CHUNK)],
                osem[b]).wait()

        def scale(b):
            src, dst = gbuf[b], obuf[b]

            def scale_rows(i, c):
                for r in range(ROW_UNROLL):
                    row = i * ROW_UNROLL + r
                    for j in range(D_MODEL // LANES):
                        sl = pl.ds(j * LANES, LANES)
                        dst[row, sl] = src[row, sl] * SCALE
                return c

            lax.fori_loop(0, CHUNK // ROW_UNROLL, scale_rows, 0)

        start_gather(0, 0)
        start_gather(1, 1)

        def body(h, carry):
            g = 2 * h
            for b in range(2):
                wait_gather(b)

                @pl.when(h > 0)
                def _():
                    wait_out(b)

                scale(b)
                start_out(g + b, b)
                start_gather((g + b + 2) % n_chunks, b)
            return carry

        lax.fori_loop(0, n_chunks // 2, body, 0)
        wait_gather(0)
        wait_gather(1)
        wait_out(0)
        wait_out(1)

    return gather_kernel


def kernel(x, table):
    n_rows = x.size
    x2d = x.reshape(n_rows // CHUNK, CHUNK)
    out = _build_sc_gather(n_rows)(x2d, table)
    return out.reshape(x.shape + (D_MODEL,))
